# baseline (speedup 1.0000x reference)
import functools

import numpy as np
import jax
import jax.numpy as jnp
from jax import lax
from jax.experimental import pallas as pl
from jax.experimental.pallas import tpu as pltpu
from jax.experimental.pallas import tpu_sc as plsc

B = 512
P = 1039
C = 64
NQ = 37
G = NQ * NQ
NUSED = 1027
ROWPAD = 17
CW = 16
CQ = C // CW
NTILES = 32
BPT = B // NTILES
UNITS = BPT * CQ
PCH = (NUSED + 15) // 16


def _grid_positions():
    dirs = [(1, 0), (0, 1), (-1, 1), (-1, 0), (0, -1), (1, -1)]
    coords = [(0, 0)]
    k = 1
    while len(coords) < P:
        q, r = 0, -k
        for d in range(6):
            for _ in range(k):
                coords.append((q, r))
                q += dirs[d][0]
                r += dirs[d][1]
        k += 1
    coords = coords[:P]
    gpos = np.zeros((PCH * 16,), dtype=np.int32)
    for p, (q, r) in enumerate(coords):
        if abs(q) <= 18 and abs(r) <= 18 and p < NUSED:
            gpos[p] = (q + 18) * NQ + (r + 18)
    return gpos


_GPOS = _grid_positions()


def _sc_kernel(hexa_hbm, gpos_hbm, zeros_hbm, out_hbm,
               x0, x1, o0, o1, gpos_v,
               isem0, isem1, osem0, osem1):
    wid = lax.axis_index("s") * 2 + lax.axis_index("c")
    iota = lax.broadcasted_iota(jnp.int32, (16,), 0)
    xb = (x0, x1)
    ob = (o0, o1)
    isem = (isem0, isem1)
    osem = (osem0, osem1)

    pltpu.sync_copy(gpos_hbm, gpos_v)
    pltpu.sync_copy(zeros_hbm, o0)
    pltpu.sync_copy(zeros_hbm, o1)

    b0 = wid * BPT

    def in_start(u, par):
        b = b0 + u // CQ
        q = u % CQ
        pltpu.async_copy(
            hexa_hbm.at[b, :, pl.ds(q * CW, CW)],
            xb[par].at[pl.ds(0, P), pl.ds(0, CW)],
            isem[par],
        )

    def in_wait(par):
        pltpu.make_async_copy(
            hexa_hbm.at[b0, :, pl.ds(0, CW)],
            xb[par].at[pl.ds(0, P), pl.ds(0, CW)],
            isem[par],
        ).wait()

    def out_start(u, par):
        b = b0 + u // CQ
        q = u % CQ
        pltpu.async_copy(ob[par], out_hbm.at[b, pl.ds(q * CW, CW), :], osem[par])

    def out_wait(par):
        pltpu.make_async_copy(ob[par], out_hbm.at[b0, pl.ds(0, CW), :],
                              osem[par]).wait()

    in_start(0, 0)

    def pair(i2, carry):
        for par in range(2):
            u = i2 * 2 + par
            in_wait(par)

            @pl.when(u < UNITS - 1)
            def _():
                in_start(u + 1, 1 - par)

            @pl.when(u >= 2)
            def _():
                out_wait(par)

            x_v = xb[par]
            o_v = ob[par]

            def pchunk(pc, carry2):
                pbase = pc * 16
                rows = pbase + iota
                gp = gpos_v[pl.ds(pbase, 16)]
                msk = rows < NUSED
                for cc in range(CW):
                    cvec = jnp.full((16,), cc, jnp.int32)
                    vals = plsc.load_gather(x_v, [rows, cvec])
                    plsc.store_scatter(o_v, [cvec, gp], vals, mask=msk)
                return carry2

            lax.fori_loop(0, PCH, pchunk, 0)
            out_start(u, par)
        return carry

    lax.fori_loop(0, UNITS // 2, pair, 0)
    out_wait(0)
    out_wait(1)


def kernel(hexa):
    mesh = plsc.VectorSubcoreMesh(core_axis_name="c", subcore_axis_name="s")
    run = functools.partial(
        pl.kernel,
        mesh=mesh,
        compiler_params=pltpu.CompilerParams(
            needs_layout_passes=False, use_tc_tiling_on_sc=False
        ),
        out_type=jax.ShapeDtypeStruct((B, C, G), jnp.float32),
        scratch_types=[
            pltpu.VMEM((PCH * 16 + 16, ROWPAD), jnp.float32),
            pltpu.VMEM((PCH * 16 + 16, ROWPAD), jnp.float32),
            pltpu.VMEM((CW, G), jnp.float32),
            pltpu.VMEM((CW, G), jnp.float32),
            pltpu.VMEM((PCH * 16,), jnp.int32),
            pltpu.SemaphoreType.DMA,
            pltpu.SemaphoreType.DMA,
            pltpu.SemaphoreType.DMA,
            pltpu.SemaphoreType.DMA,
        ],
    )(_sc_kernel)
    gpos = jnp.asarray(_GPOS)
    zeros = jnp.zeros((CW, G), jnp.float32)
    out = run(hexa, gpos, zeros)
    return out.reshape(B, C, NQ, NQ)
